# baseline (speedup 1.0000x reference)
import functools

import jax
import jax.numpy as jnp
from jax import lax
from jax.experimental import pallas as pl
from jax.experimental.pallas import tpu as pltpu
from jax.experimental.pallas import tpu_sc as plsc

VOCAB = 1000000
DIM = 64
BATCH = 16384
HIST = 20

NC, NS = 2, 16
NW = NC * NS

GRP = 128
NBLK = BATCH // GRP
K_PER_W = NBLK // NW
U_PER_W = HIST * K_PER_W
NPAIR = U_PER_W // 2


def _build():
    mesh = plsc.VectorSubcoreMesh(core_axis_name="c", subcore_axis_name="s")

    @functools.partial(
        pl.kernel,
        mesh=mesh,
        out_type=jax.ShapeDtypeStruct((HIST, 8, NBLK, 8, GRP), jnp.float32),
        scratch_types=[
            pltpu.VMEM((U_PER_W, GRP), jnp.int32),
            pltpu.VMEM((2, GRP, DIM), jnp.float32),
            pltpu.VMEM((2, 8, 8, GRP), jnp.float32),
            pltpu.SemaphoreType.DMA,
            pltpu.SemaphoreType.DMA,
            pltpu.SemaphoreType.DMA,
            pltpu.SemaphoreType.DMA,
            pltpu.SemaphoreType.DMA,
        ],
        compiler_params=pltpu.CompilerParams(
            use_tc_tiling_on_sc=False, needs_layout_passes=False
        ),
    )
    def gather_kernel(idx_hbm, table_hbm, out_hbm, idx_v, rows_v, trans_v,
                      isem, g0, g1, w0, w1):
        wid = lax.axis_index("s") * NC + lax.axis_index("c")
        gsem = (g0, g1)
        wsem = (w0, w1)
        lanes = lax.iota(jnp.int32, 16)

        def unit_hk(u):
            return lax.shift_right_logical(u, 2), lax.bitwise_and(u, 3)

        def idx_load(u, carry):
            h, k = unit_hk(u)
            pltpu.async_copy(
                idx_hbm.at[h, pl.ds((wid * K_PER_W + k) * GRP, GRP)],
                idx_v.at[u], isem,
            )
            return carry

        def idx_drain(u, carry):
            pltpu.make_async_copy(
                idx_hbm.at[0, pl.ds(0, GRP)], idx_v.at[0], isem
            ).wait()
            return carry

        lax.fori_loop(0, U_PER_W, idx_load, 0)
        lax.fori_loop(0, U_PER_W, idx_drain, 0)

        def gather_start(u, buf):
            pltpu.async_copy(table_hbm.at[idx_v.at[u]], rows_v.at[buf], gsem[buf])

        def gather_wait(buf):
            pltpu.make_async_copy(
                table_hbm.at[pl.ds(0, GRP)], rows_v.at[buf], gsem[buf]
            ).wait()

        def write_start(u, buf):
            h, k = unit_hk(u)
            pltpu.async_copy(
                trans_v.at[buf], out_hbm.at[h, :, wid * K_PER_W + k], wsem[buf]
            )

        def write_wait(buf):
            pltpu.make_async_copy(
                trans_v.at[buf], out_hbm.at[0, :, 0], wsem[buf]
            ).wait()

        def transpose(buf):
            rows = rows_v.at[buf]
            trans = trans_v.at[buf]

            def cb_body(cb, carry):
                ri = pl.multiple_of(cb * 16, 16) + lanes
                for d in range(DIM):
                    ci = jnp.full((16,), d, jnp.int32)
                    vals = plsc.load_gather(rows, [ri, ci])
                    trans[d // 8, d % 8, pl.ds(pl.multiple_of(cb * 16, 16), 16)] = vals
                return carry

            lax.fori_loop(0, GRP // 16, cb_body, 0)

        gather_start(0, 0)

        def body(i, carry):
            u0 = 2 * i
            gather_start(u0 + 1, 1)
            gather_wait(0)

            @pl.when(i > 0)
            def _w0():
                write_wait(0)

            transpose(0)
            write_start(u0, 0)

            @pl.when(i < NPAIR - 1)
            def _g0():
                gather_start(u0 + 2, 0)

            gather_wait(1)

            @pl.when(i > 0)
            def _w1():
                write_wait(1)

            transpose(1)
            write_start(u0 + 1, 1)
            return carry

        lax.fori_loop(0, NPAIR, body, 0)
        write_wait(0)
        write_wait(1)

    return gather_kernel


_gather = _build()


def kernel(indices, weight):
    wpad = jnp.pad(weight, ((0, 0), (0, 128 - DIM)))
    table = wpad.reshape(2 * VOCAB, DIM)
    idx2 = indices.T.astype(jnp.int32) * 2
    t = _gather(idx2, table)
    return t.transpose(2, 4, 0, 1, 3).reshape(BATCH, HIST, DIM)
